# baseline (speedup 1.0000x reference)
import functools

import jax
import jax.numpy as jnp
from jax import lax
from jax.experimental import pallas as pl
from jax.experimental.pallas import tpu as pltpu
from jax.experimental.pallas import tpu_sc as plsc


def kernel(g, h, r, norm, W):
    B = h.shape[0]
    V, D = W.shape
    L = 16

    info = plsc.get_sparse_core_info()
    NC, NS = info.num_cores, info.num_subcores
    NW = NC * NS
    NBLK_T = (V + 127) // 128
    CB = 6
    CHW = CB * 128
    NP = 42
    MAXPB = NBLK_T - CB
    FLUSH_CAP = 128
    TRASH0 = B

    idx = h.reshape(B)
    WT = W.T

    mesh = plsc.VectorSubcoreMesh(core_axis_name="c", subcore_axis_name="s")

    @functools.partial(
        pl.kernel,
        mesh=mesh,
        compiler_params=pltpu.CompilerParams(
            use_tc_tiling_on_sc=True, needs_layout_passes=False),
        out_type=jax.ShapeDtypeStruct((B + NW, 128), jnp.float32),
        scratch_types=[
            pltpu.VMEM((B,), jnp.int32),
            pltpu.VMEM((B + L,), jnp.int32),
            pltpu.VMEM((B + L,), jnp.int32),
            pltpu.VMEM((2, D, CHW), jnp.float32),
            pltpu.VMEM((FLUSH_CAP, 128), jnp.float32),
            pltpu.VMEM((1, FLUSH_CAP), jnp.int32),
            [pltpu.SemaphoreType.DMA] * 3,
        ],
    )
    def gather_kernel(idx_hbm, wt_hbm, outp_hbm,
                      idxall_v, comp_r, comp_b, chunk_v, rows_v, blist_v,
                      sems):
        wid = lax.axis_index("s") * NC + lax.axis_index("c")
        lo_blk = wid * NBLK_T // NW
        hi_blk = (wid + 1) * NBLK_T // NW
        lo_col = lo_blk * 128
        hi_col = hi_blk * 128
        lanes = lax.iota(jnp.int32, L)
        zeros = jnp.zeros((L,), jnp.int32)

        pltpu.sync_copy(idx_hbm, idxall_v)

        trash = jnp.full((L,), TRASH0, jnp.int32) + jnp.broadcast_to(wid, (L,))
        for q in range(FLUSH_CAP // L):
            plsc.store_scatter(blist_v, [zeros, lanes + q * L], trash)

        def _issue(p, buf):
            pb = lax.min(lo_blk + p * CB, jnp.int32(MAXPB))
            off = pl.multiple_of(pb * 128, 128)
            return pltpu.async_copy(
                wt_hbm.at[:, pl.ds(off, CHW)], chunk_v.at[buf], sems[buf])

        _issue(0, 0)
        _issue(1, 1)

        lo_v = jnp.broadcast_to(lo_col, (L,)).astype(jnp.int32)
        hi_v = jnp.broadcast_to(hi_col, (L,)).astype(jnp.int32)

        def compact(gi, cnt_v):
            r_vec = idxall_v[pl.ds(gi * L, L)]
            b_vec = lanes + gi * L
            m = jnp.logical_and(r_vec >= lo_v, r_vec < hi_v)
            pm = plsc.cumsum(m.astype(jnp.int32))
            pos = cnt_v + pm - 1
            plsc.store_scatter(comp_r, [pos], r_vec, mask=m)
            plsc.store_scatter(comp_b, [pos], b_vec, mask=m)
            return cnt_v + jnp.broadcast_to(pm[L - 1], (L,))

        cnt_v = lax.fori_loop(0, B // L, compact, zeros)
        nh = cnt_v[0]
        ng = lax.div(nh + (L - 1), jnp.int32(L))
        nh_v = jnp.broadcast_to(nh, (L,))

        def flush(rcnt_v):
            pltpu.async_copy(
                rows_v, outp_hbm.at[blist_v.at[0]], sems[2]).wait()
            return rcnt_v

        def extract_pass(p, buf, rcnt_v):
            pb = lax.min(lo_blk + p * CB, jnp.int32(MAXPB))
            off_v = jnp.broadcast_to(pb * 128, (L,)).astype(jnp.int32)
            buf_v = jnp.broadcast_to(buf, (L,)).astype(jnp.int32)

            def grp(g2, rcnt_v):
                r_vec = comp_r[pl.ds(g2 * L, L)]
                b_vec = comp_b[pl.ds(g2 * L, L)]
                valid = (lanes + g2 * L) < nh_v
                m = jnp.logical_and(
                    valid,
                    jnp.logical_and(r_vec >= off_v,
                                    r_vec < off_v + CHW))
                npc = plsc.all_reduce_population_count(m)

                def do_extract(rcnt_v):
                    rcnt_v = lax.cond(
                        rcnt_v[0] + L > FLUSH_CAP,
                        lambda rv: flush(rv) * 0,
                        lambda rv: rv,
                        rcnt_v)
                    rel = r_vec - off_v
                    pm = plsc.cumsum(m.astype(jnp.int32))
                    pos = rcnt_v + pm - 1
                    plsc.store_scatter(blist_v, [zeros, pos], b_vec, mask=m)
                    for d in range(D):
                        d_v = jnp.full((L,), d, jnp.int32)
                        vals = plsc.load_gather(
                            chunk_v, [buf_v, d_v, rel], mask=m)
                        plsc.store_scatter(rows_v, [pos, d_v], vals, mask=m)
                    return rcnt_v + jnp.broadcast_to(pm[L - 1], (L,))

                return lax.cond(npc[0] > 0, do_extract,
                                lambda rv: rv, rcnt_v)

            return lax.fori_loop(0, ng, grp, rcnt_v)

        def pass_pair(p2, rcnt_v):
            pa = p2 * 2
            pltpu.make_async_copy(
                wt_hbm.at[:, pl.ds(0, CHW)], chunk_v.at[0], sems[0]).wait()
            rcnt_v = extract_pass(pa, 0, rcnt_v)
            _issue(pa + 2, 0)
            pltpu.make_async_copy(
                wt_hbm.at[:, pl.ds(0, CHW)], chunk_v.at[1], sems[1]).wait()
            rcnt_v = extract_pass(pa + 1, 1, rcnt_v)
            _issue(pa + 3, 1)
            return rcnt_v

        rcnt_v = lax.fori_loop(0, NP // 2 - 1, pass_pair, zeros)
        pltpu.make_async_copy(
            wt_hbm.at[:, pl.ds(0, CHW)], chunk_v.at[0], sems[0]).wait()
        rcnt_v = extract_pass(NP - 2, 0, rcnt_v)
        pltpu.make_async_copy(
            wt_hbm.at[:, pl.ds(0, CHW)], chunk_v.at[1], sems[1]).wait()
        rcnt_v = extract_pass(NP - 1, 1, rcnt_v)
        flush(rcnt_v)

    out_pad = gather_kernel(idx, WT)
    return out_pad[:B, :D]


# speedup vs baseline: 1.1821x; 1.1821x over previous
import functools

import jax
import jax.numpy as jnp
from jax import lax
from jax.experimental import pallas as pl
from jax.experimental.pallas import tpu as pltpu
from jax.experimental.pallas import tpu_sc as plsc


def kernel(g, h, r, norm, W):
    B = h.shape[0]
    V, D = W.shape
    L = 16

    info = plsc.get_sparse_core_info()
    NC, NS = info.num_cores, info.num_subcores
    NW = NC * NS
    bpw = B // NW
    K = 16
    NBLK = bpw // K

    idx = h.reshape(B)
    WT = W.T

    mesh = plsc.VectorSubcoreMesh(core_axis_name="c", subcore_axis_name="s")

    def _block_copy(wt_hbm, idx_vec, j, blk_v, sem):
        off = pl.multiple_of(
            lax.shift_left(lax.shift_right_logical(idx_vec[j], 7), 7), 128)
        for gg in range(D // 8):
            pltpu.async_copy(
                wt_hbm.at[pl.ds(gg * 8, 8), pl.ds(off, 128)],
                blk_v.at[j, pl.ds(gg * 8, 8)], sem)

    @functools.partial(
        pl.kernel,
        mesh=mesh,
        compiler_params=pltpu.CompilerParams(
            use_tc_tiling_on_sc=True, needs_layout_passes=False),
        out_type=jax.ShapeDtypeStruct((D, B), jnp.float32),
        scratch_types=[
            pltpu.VMEM((bpw,), jnp.int32),
            pltpu.VMEM((K, D, 128), jnp.float32),
            pltpu.VMEM((D, bpw), jnp.float32),
            [pltpu.SemaphoreType.DMA] * K,
        ],
    )
    def gather_kernel(idx_hbm, wt_hbm, outt_hbm, idx_v, blk_v, outt_v, sems):
        wid = lax.axis_index("s") * NC + lax.axis_index("c")
        base = wid * bpw
        pltpu.sync_copy(idx_hbm.at[pl.ds(base, bpw)], idx_v)
        lanes = lax.iota(jnp.int32, L)

        idx_vec0 = idx_v[pl.ds(0, K)]
        for j in range(K):
            _block_copy(wt_hbm, idx_vec0, j, blk_v, sems[j])

        def extract(i, j, rm_vec):
            b_splat = jnp.full((L,), i * K + j, jnp.int32)
            j_splat = jnp.full((L,), j, jnp.int32)
            rm_splat = jnp.broadcast_to(rm_vec[j], (L,))
            for half in range(2):
                d_vec = lanes + half * L
                vals = plsc.load_gather(blk_v, [j_splat, d_vec, rm_splat])
                plsc.store_scatter(outt_v, [d_vec, b_splat], vals)

        def body(i, _):
            idx_vec = idx_v[pl.ds(i * K, K)]
            nxt_vec = idx_v[pl.ds((i + 1) * K, K)]
            rm_vec = lax.bitwise_and(idx_vec, 127)
            for j in range(K):
                for gg in range(D // 8):
                    pltpu.make_async_copy(
                        wt_hbm.at[pl.ds(0, 8), pl.ds(0, 128)],
                        blk_v.at[j, pl.ds(gg * 8, 8)], sems[j]).wait()
                extract(i, j, rm_vec)
                _block_copy(wt_hbm, nxt_vec, j, blk_v, sems[j])
            return 0

        lax.fori_loop(0, NBLK - 1, body, 0)

        i_last = NBLK - 1
        idx_vec = idx_v[pl.ds(i_last * K, K)]
        rm_vec = lax.bitwise_and(idx_vec, 127)
        for j in range(K):
            for gg in range(D // 8):
                pltpu.make_async_copy(
                    wt_hbm.at[pl.ds(0, 8), pl.ds(0, 128)],
                    blk_v.at[j, pl.ds(gg * 8, 8)], sems[j]).wait()
            extract(i_last, j, rm_vec)

        pltpu.sync_copy(outt_v, outt_hbm.at[:, pl.ds(base, bpw)])

    return gather_kernel(idx, WT).T
